# baseline (speedup 1.0000x reference)
import functools

import jax
import jax.numpy as jnp
from jax import lax
from jax.experimental import pallas as pl
from jax.experimental.pallas import tpu as pltpu

N_NODES = 10000
N_PAD = 10016
N_REL = 200
E_PAD = 200704
ROWS_BLK = 400

D1 = 208
D2 = 400


def _l2n(x):
    sq = jnp.sum(x * x, axis=1, keepdims=True)
    return x * lax.rsqrt(jnp.maximum(sq, 1e-12))


def _elu(x):
    return jnp.where(x > 0, x, jnp.exp(jnp.minimum(x, 0.0)) - 1.0)


def _tcA_body(ent_ref, w1_ref, a2_ref, sw_ref, sb_ref, xsd_ref, sc_ref, skip_ref):
    nrm = _l2n(ent_ref[...])
    xsd = jnp.dot(nrm, w1_ref[...], preferred_element_type=jnp.float32)
    xsd_ref[...] = xsd
    sc_ref[...] = jnp.dot(xsd, a2_ref[...], preferred_element_type=jnp.float32)
    skip_ref[...] = jnp.dot(nrm, sw_ref[...], preferred_element_type=jnp.float32) + sb_ref[...]


def _tc_prep1(entity_emb, w1cat, a2blk, skip_w, skip_b2):
    grid = (N_NODES // ROWS_BLK,)
    return pl.pallas_call(
        _tcA_body,
        grid=grid,
        in_specs=[
            pl.BlockSpec((ROWS_BLK, 128), lambda i: (i, 0)),
            pl.BlockSpec((128, 400), lambda i: (0, 0)),
            pl.BlockSpec((400, 8), lambda i: (0, 0)),
            pl.BlockSpec((128, 200), lambda i: (0, 0)),
            pl.BlockSpec((1, 200), lambda i: (0, 0)),
        ],
        out_specs=[
            pl.BlockSpec((ROWS_BLK, 400), lambda i: (i, 0)),
            pl.BlockSpec((ROWS_BLK, 8), lambda i: (i, 0)),
            pl.BlockSpec((ROWS_BLK, 200), lambda i: (i, 0)),
        ],
        out_shape=[
            jax.ShapeDtypeStruct((N_NODES, 400), jnp.float32),
            jax.ShapeDtypeStruct((N_NODES, 8), jnp.float32),
            jax.ShapeDtypeStruct((N_NODES, 200), jnp.float32),
        ],
    )(entity_emb, w1cat, a2blk, skip_w, skip_b2)


def _tcR_body(rel_ref, wr1_ref, wr2_ref, wp1_ref, wp2_ref, a2r_ref, b2r_ref,
              rel2_ref, relp1_ref, relp2_ref, rsc1_ref, rsc2_ref):
    rel = rel_ref[...]
    r1 = jnp.dot(rel, wr1_ref[...], preferred_element_type=jnp.float32)
    rel2_ref[...] = jnp.dot(r1, wr2_ref[...], preferred_element_type=jnp.float32)
    relp1 = jnp.dot(rel, wp1_ref[...], preferred_element_type=jnp.float32)
    relp2 = jnp.dot(r1, wp2_ref[...], preferred_element_type=jnp.float32)
    relp1_ref[...] = relp1
    relp2_ref[...] = relp2
    rsc1_ref[...] = jnp.dot(relp1, a2r_ref[...], preferred_element_type=jnp.float32)
    rsc2_ref[...] = jnp.dot(relp2, b2r_ref[...], preferred_element_type=jnp.float32)


def _tc_rel(relation_emb, w_rel1, w_rel2, wp1, wp2, a2r, b2r):
    full = lambda shp: pl.BlockSpec(shp, lambda: tuple(0 for _ in shp))
    return pl.pallas_call(
        _tcR_body,
        in_specs=[full((200, 64)), full((64, 100)), full((100, 200)),
                  full((64, D1)), full((100, D2)), full((D1, 8)), full((D2, 8))],
        out_specs=[full((200, 200)), full((200, D1)), full((200, D2)),
                   full((200, 8)), full((200, 8))],
        out_shape=[
            jax.ShapeDtypeStruct((200, 200), jnp.float32),
            jax.ShapeDtypeStruct((200, D1), jnp.float32),
            jax.ShapeDtypeStruct((200, D2), jnp.float32),
            jax.ShapeDtypeStruct((200, 8), jnp.float32),
            jax.ShapeDtypeStruct((200, 8), jnp.float32),
        ],
    )(relation_emb, w_rel1, w_rel2, wp1, wp2, a2r, b2r)


def _tcB_body(acc_ref, xsd1_ref, w2_ref, b2_ref, xsd2_ref, sc2_ref):
    acc = acc_ref[...]
    rs0 = acc[:, 100:101]
    rs1 = acc[:, 204:205]
    xs0 = xsd1_ref[:, 0:100]
    xs1 = xsd1_ref[:, 200:300]
    h0 = (rs0 * xs0 + acc[:, 0:100]) / jnp.where(rs0 == 0.0, 1e-12, rs0)
    h1 = (rs1 * xs1 + acc[:, 104:204]) / jnp.where(rs1 == 0.0, 1e-12, rs1)
    out1 = jnp.concatenate([_elu(h0), _elu(h1)], axis=1)
    xsd2 = jnp.dot(out1, w2_ref[...], preferred_element_type=jnp.float32)
    xsd2_ref[...] = xsd2
    sc2_ref[...] = jnp.dot(xsd2, b2_ref[...], preferred_element_type=jnp.float32)


def _tc_mid(acc1, xsd1, w2cat, b2blk):
    grid = (N_NODES // ROWS_BLK,)
    return pl.pallas_call(
        _tcB_body,
        grid=grid,
        in_specs=[
            pl.BlockSpec((ROWS_BLK, D1), lambda i: (i, 0)),
            pl.BlockSpec((ROWS_BLK, 400), lambda i: (i, 0)),
            pl.BlockSpec((200, 800), lambda i: (0, 0)),
            pl.BlockSpec((800, 8), lambda i: (0, 0)),
        ],
        out_specs=[
            pl.BlockSpec((ROWS_BLK, 800), lambda i: (i, 0)),
            pl.BlockSpec((ROWS_BLK, 8), lambda i: (i, 0)),
        ],
        out_shape=[
            jax.ShapeDtypeStruct((N_NODES, 800), jnp.float32),
            jax.ShapeDtypeStruct((N_NODES, 8), jnp.float32),
        ],
    )(acc1, xsd1, w2cat, b2blk)


def _tcC_body(acc_ref, rs_ref, xsd2_ref, skip_ref, batch_ref, out_ref):
    i = pl.program_id(0)
    acc = acc_ref[...]
    rs0 = rs_ref[:, 0:1]
    rs1 = rs_ref[:, 1:2]
    xs0 = xsd2_ref[:, 0:200]
    xs1 = xsd2_ref[:, 400:600]
    h0 = (rs0 * xs0 + acc[:, 0:200]) / jnp.where(rs0 == 0.0, 1e-12, rs0)
    h1 = (rs1 * xs1 + acc[:, 200:400]) / jnp.where(rs1 == 0.0, 1e-12, rs1)
    out2 = _elu((h0 + h1) * 0.5)
    rows = i * ROWS_BLK + lax.broadcasted_iota(jnp.int32, (ROWS_BLK, 1024), 0)
    eq = (rows == batch_ref[...]).astype(jnp.float32)
    mask = jnp.max(eq, axis=1, keepdims=True)
    out_ref[...] = _l2n(skip_ref[...] + mask * out2)


def _tc_final(acc2, rowsum2, xsd2, skip, batch_row):
    grid = (N_NODES // ROWS_BLK,)
    return pl.pallas_call(
        _tcC_body,
        grid=grid,
        in_specs=[
            pl.BlockSpec((ROWS_BLK, D2), lambda i: (i, 0)),
            pl.BlockSpec((ROWS_BLK, 8), lambda i: (i, 0)),
            pl.BlockSpec((ROWS_BLK, 800), lambda i: (i, 0)),
            pl.BlockSpec((ROWS_BLK, 200), lambda i: (i, 0)),
            pl.BlockSpec((1, 1024), lambda i: (0, 0)),
        ],
        out_specs=pl.BlockSpec((ROWS_BLK, 200), lambda i: (i, 0)),
        out_shape=jax.ShapeDtypeStruct((N_NODES, 200), jnp.float32),
    )(acc2, rowsum2, xsd2, skip, batch_row)


def _edge_phase_xla(src, dst, t1, t2, nsc, rsc, xdr, relp, dpad, bc):
    def head(h):
        arg = nsc[h, src] + nsc[2 + h, dst] + rsc[h, t1] + rsc[h, t2]
        return jnp.exp(-jnp.where(arg >= 0, arg, 0.2 * arg))
    e0 = head(0)
    e1 = head(1)
    col = jnp.arange(dpad)
    w = jnp.where(col[None, :] < bc * 16 + 8, e0[:, None], e1[:, None])
    rows = xdr[dst] + relp[t1] + relp[t2]
    acc = jax.ops.segment_sum(w * rows, src, num_segments=N_PAD)[:N_NODES]
    rs = jax.ops.segment_sum(jnp.stack([e0, e1], axis=1), src, num_segments=N_PAD)[:N_NODES]
    rs = jnp.concatenate([rs, jnp.zeros((N_NODES, 6), jnp.float32)], axis=1)
    return acc, rs


def kernel(entity_emb, relation_emb, edges, edge_types, batch_inputs, train_indices_nhop,
           a1_0, a1_1, a2_0, a2_1, w_rel1, b1_0, b1_1, b2_0, b2_1, w_rel2, skip_w, skip_b):
    f32 = jnp.float32
    npad = E_PAD - 160000 - 40000
    src = jnp.concatenate([edges[0], train_indices_nhop[:, 3],
                           jnp.full((npad,), N_NODES, jnp.int32)])
    dst = jnp.concatenate([edges[1], train_indices_nhop[:, 0],
                           jnp.zeros((npad,), jnp.int32)])
    t1 = jnp.concatenate([edge_types, train_indices_nhop[:, 1],
                          jnp.full((npad,), N_REL, jnp.int32)])
    t2 = jnp.concatenate([jnp.full((160000,), N_REL, jnp.int32), train_indices_nhop[:, 2],
                          jnp.full((npad,), N_REL, jnp.int32)])

    w1cat = jnp.concatenate([a1_0[:, :128].T, a1_0[:, 128:256].T,
                             a1_1[:, :128].T, a1_1[:, 128:256].T], axis=1)
    z100 = jnp.zeros((100,), f32)
    a2blk = jnp.zeros((400, 8), f32)
    a2blk = a2blk.at[0:100, 0].set(a2_0[0]).at[100:200, 1].set(a2_0[0])
    a2blk = a2blk.at[200:300, 2].set(a2_1[0]).at[300:400, 3].set(a2_1[0])
    wp1 = jnp.zeros((64, D1), f32)
    wp1 = wp1.at[:, 0:100].set(a1_0[:, 256:].T).at[:, 104:204].set(a1_1[:, 256:].T)
    a2r = jnp.zeros((D1, 8), f32)
    a2r = a2r.at[0:100, 0].set(a2_0[0]).at[104:204, 1].set(a2_1[0])
    w2cat = jnp.concatenate([b1_0[:, :200].T, b1_0[:, 200:400].T,
                             b1_1[:, :200].T, b1_1[:, 200:400].T], axis=1)
    b2blk = jnp.zeros((800, 8), f32)
    b2blk = b2blk.at[0:200, 0].set(b2_0[0]).at[200:400, 1].set(b2_0[0])
    b2blk = b2blk.at[400:600, 2].set(b2_1[0]).at[600:800, 3].set(b2_1[0])
    wp2 = jnp.zeros((100, D2), f32)
    wp2 = wp2.at[:, 0:200].set(b1_0[:, 400:].T).at[:, 200:400].set(b1_1[:, 400:].T)
    b2r = jnp.zeros((D2, 8), f32)
    b2r = b2r.at[0:200, 0].set(b2_0[0]).at[200:400, 1].set(b2_1[0])

    xsd1, sc1, skip = _tc_prep1(entity_emb, w1cat, a2blk, skip_w, skip_b[None, :])
    rel2, relp1, relp2, rsc1, rsc2 = _tc_rel(relation_emb, w_rel1, w_rel2, wp1, wp2, a2r, b2r)

    nsc1 = jnp.pad(jnp.stack([sc1[:, 0], sc1[:, 2], sc1[:, 1], sc1[:, 3]]),
                   ((0, 0), (0, N_PAD - N_NODES)))
    rsc1t = jnp.pad(rsc1[:, :2].T, ((0, 0), (0, 208 - 200)))
    ones = jnp.ones((N_NODES, 1), f32)
    z3 = jnp.zeros((N_NODES, 3), f32)
    xdr1 = jnp.concatenate([xsd1[:, 100:200], ones, z3, xsd1[:, 300:400], ones, z3], axis=1)
    xdr1 = jnp.pad(xdr1, ((0, N_PAD - N_NODES), (0, 0)))
    relp1e = jnp.pad(relp1, ((0, 1), (0, 0)))

    acc1, _ = _edge_phase_xla(src, dst, t1, t2, nsc1, rsc1t, xdr1, relp1e, D1, 6)

    xsd2, sc2 = _tc_mid(acc1, xsd1, w2cat, b2blk)

    nsc2 = jnp.pad(jnp.stack([sc2[:, 0], sc2[:, 2], sc2[:, 1], sc2[:, 3]]),
                   ((0, 0), (0, N_PAD - N_NODES)))
    rsc2t = jnp.pad(rsc2[:, :2].T, ((0, 0), (0, 208 - 200)))
    xdr2 = jnp.concatenate([xsd2[:, 200:400], xsd2[:, 600:800]], axis=1)
    xdr2 = jnp.pad(xdr2, ((0, N_PAD - N_NODES), (0, 0)))
    relp2e = jnp.pad(relp2, ((0, 1), (0, 0)))

    acc2, rs2 = _edge_phase_xla(src, dst, t1, t2, nsc2, rsc2t, xdr2, relp2e, D2, 12)

    out_entity = _tc_final(acc2, rs2, xsd2, skip, batch_inputs[:, 2][None, :])
    return out_entity, rel2
